# baseline (speedup 1.0000x reference)
import functools

import jax
import jax.numpy as jnp
from jax import lax
from jax.experimental import pallas as pl
from jax.experimental.pallas import tpu as pltpu


def _round_up(x, m):
    return (x + m - 1) // m * m


def _mf_gather_kernel(block_b, nsteps,
                      u_idx_ref, v_idx_ref,
                      user_tab_hbm, item_tab_hbm,
                      out_ref,
                      u_rows, v_rows,
                      sems):
    c = pl.program_id(0)
    g = pl.program_id(1)

    def issue(step, slot):
        base = (c * nsteps + step) * block_b
        for r in range(block_b):
            ui = u_idx_ref[base + r]
            vi = v_idx_ref[base + r]
            u8 = pl.multiple_of((ui >> 3) << 3, 8)
            v8 = pl.multiple_of((vi >> 3) << 3, 8)
            pltpu.async_copy(user_tab_hbm.at[pl.ds(u8, 8)],
                             u_rows.at[slot, pl.ds(r * 8, 8)],
                             sems.at[slot, 0], priority=0)
            pltpu.async_copy(item_tab_hbm.at[pl.ds(v8, 8)],
                             v_rows.at[slot, pl.ds(r * 8, 8)],
                             sems.at[slot, 1], priority=1)

    def wait_slot(slot):
        pltpu.make_async_copy(user_tab_hbm.at[pl.ds(0, block_b * 8)],
                              u_rows.at[slot], sems.at[slot, 0]).wait()
        pltpu.make_async_copy(item_tab_hbm.at[pl.ds(0, block_b * 8)],
                              v_rows.at[slot], sems.at[slot, 1]).wait()

    cur = lax.rem(g, 2)

    @pl.when(g == 0)
    def _():
        issue(0, 0)

    @pl.when(g + 1 < nsteps)
    def _():
        issue(g + 1, lax.rem(g + 1, 2))

    wait_slot(cur)

    w = u_rows[cur, pl.ds(0, block_b)] * v_rows[cur, pl.ds(0, block_b)]
    out_ref[...] = jnp.sum(w, axis=1, keepdims=True)


def kernel(u, v, user_tab, item_tab):
    B = u.shape[0]
    ep = user_tab.shape[1]
    ncores = 2
    block_b = 512

    per_core = _round_up(pl.cdiv(B, ncores), block_b)
    nsteps = per_core // block_b
    b_pad = ncores * per_core

    u_idx = jnp.zeros((b_pad,), jnp.int32).at[:B].set(
        u.astype(jnp.int32).reshape(B))
    v_idx = jnp.zeros((b_pad,), jnp.int32).at[:B].set(
        v.astype(jnp.int32).reshape(B))

    grid_spec = pltpu.PrefetchScalarGridSpec(
        num_scalar_prefetch=2,
        grid=(ncores, nsteps),
        in_specs=[pl.BlockSpec(memory_space=pl.ANY),
                  pl.BlockSpec(memory_space=pl.ANY)],
        out_specs=pl.BlockSpec((block_b, 1),
                               lambda c, g, u_ref, v_ref: (c * nsteps + g, 0)),
        scratch_shapes=[
            pltpu.VMEM((2, block_b * 8, ep), jnp.float32),
            pltpu.VMEM((2, block_b * 8, ep), jnp.float32),
            pltpu.SemaphoreType.DMA((2, 2)),
        ],
    )
    out = pl.pallas_call(
        functools.partial(_mf_gather_kernel, block_b, nsteps),
        out_shape=jax.ShapeDtypeStruct((b_pad, 1), jnp.float32),
        grid_spec=grid_spec,
        compiler_params=pltpu.CompilerParams(
            dimension_semantics=("parallel", "arbitrary"),
            disable_bounds_checks=True),
    )(u_idx, v_idx, user_tab, item_tab)
    return out[:B, 0]


# speedup vs baseline: 1.0112x; 1.0112x over previous
import functools

import jax
import jax.numpy as jnp
from jax import lax
from jax.experimental import pallas as pl
from jax.experimental.pallas import tpu as pltpu


def _round_up(x, m):
    return (x + m - 1) // m * m


def _mf_gather_kernel(block_b, nsteps,
                      u_idx_ref, v_idx_ref,
                      user_tab_hbm, item_tab_hbm,
                      out_ref,
                      u_rows, v_rows,
                      sems):
    c = pl.program_id(0)
    g = pl.program_id(1)

    def issue(step, slot):
        base = (c * nsteps + step) * block_b
        for r in range(block_b):
            ui = u_idx_ref[base + r]
            vi = v_idx_ref[base + r]
            pltpu.async_copy(user_tab_hbm.at[ui], u_rows.at[slot, r],
                             sems.at[slot, 0], priority=0)
            pltpu.async_copy(item_tab_hbm.at[vi], v_rows.at[slot, r],
                             sems.at[slot, 1], priority=1)

    def wait_slot(slot):
        pltpu.make_async_copy(user_tab_hbm.at[pl.ds(0, block_b)],
                              u_rows.at[slot], sems.at[slot, 0]).wait()
        pltpu.make_async_copy(item_tab_hbm.at[pl.ds(0, block_b)],
                              v_rows.at[slot], sems.at[slot, 1]).wait()

    cur = lax.rem(g, 2)

    @pl.when(g == 0)
    def _():
        issue(0, 0)

    @pl.when(g + 1 < nsteps)
    def _():
        issue(g + 1, lax.rem(g + 1, 2))

    wait_slot(cur)

    w = u_rows[cur] * v_rows[cur]
    out_ref[...] = jnp.sum(w, axis=1, keepdims=True)


def kernel(u, v, user_tab, item_tab):
    B = u.shape[0]
    ep = user_tab.shape[1]
    ncores = 2
    block_b = 512

    per_core = _round_up(pl.cdiv(B, ncores), block_b)
    nsteps = per_core // block_b
    b_pad = ncores * per_core

    u_idx = jnp.zeros((b_pad,), jnp.int32).at[:B].set(
        u.astype(jnp.int32).reshape(B))
    v_idx = jnp.zeros((b_pad,), jnp.int32).at[:B].set(
        v.astype(jnp.int32).reshape(B))

    grid_spec = pltpu.PrefetchScalarGridSpec(
        num_scalar_prefetch=2,
        grid=(ncores, nsteps),
        in_specs=[pl.BlockSpec(memory_space=pl.ANY),
                  pl.BlockSpec(memory_space=pl.ANY)],
        out_specs=pl.BlockSpec((block_b, 1),
                               lambda c, g, u_ref, v_ref: (c * nsteps + g, 0)),
        scratch_shapes=[
            pltpu.VMEM((2, block_b, ep), jnp.float32),
            pltpu.VMEM((2, block_b, ep), jnp.float32),
            pltpu.SemaphoreType.DMA((2, 2)),
        ],
    )
    out = pl.pallas_call(
        functools.partial(_mf_gather_kernel, block_b, nsteps),
        out_shape=jax.ShapeDtypeStruct((b_pad, 1), jnp.float32),
        grid_spec=grid_spec,
        compiler_params=pltpu.CompilerParams(
            dimension_semantics=("parallel", "arbitrary"),
            disable_bounds_checks=True),
    )(u_idx, v_idx, user_tab, item_tab)
    return out[:B, 0]
